# baseline (speedup 1.0000x reference)
import functools

import jax
import jax.numpy as jnp
from jax import lax
from jax.experimental import pallas as pl
from jax.experimental.pallas import tpu as pltpu
from jax.experimental.pallas import tpu_sc as plsc

N = 10000
E = 320000
D = 128
DE = 16
H = 64

NUM_CORES = 2
NUM_SUBCORES = 16
EPT = E // NUM_SUBCORES
NP = 10240
ROWS_PER_TILE = NP // NUM_SUBCORES

_HI = lax.Precision.HIGHEST


def _edge_lin_kernel(ea_ref, wl_ref, bl_ref, e_ref):
    e_ref[...] = jnp.dot(ea_ref[...], wl_ref[0]) + bl_ref[0]


def _edge_lin(edge_attr, Wl, bl, dout):
    BE = 8000
    nblk = E // BE
    dh = dout // 2
    wls = Wl.reshape(DE, 2, dh).transpose(1, 0, 2)
    bls = bl.reshape(2, 1, dh)
    return pl.pallas_call(
        _edge_lin_kernel,
        grid=(2, nblk),
        in_specs=[
            pl.BlockSpec((BE, DE), lambda j, i: (i, 0)),
            pl.BlockSpec((1, DE, dh), lambda j, i: (j, 0, 0)),
            pl.BlockSpec((1, 1, dh), lambda j, i: (j, 0, 0)),
        ],
        out_specs=pl.BlockSpec((BE, dh), lambda j, i: (j * nblk + i, 0)),
        out_shape=jax.ShapeDtypeStruct((2 * E, dh), jnp.float32),
    )(edge_attr, wls, bls)


def _node_mlp_kernel(h_ref, p0_ref, p1_ref, wa_ref, ba_ref, wb_ref, bb_ref,
                     out_ref):
    z = h_ref[...] + jnp.concatenate([p0_ref[...], p1_ref[...]], axis=1)
    t = jnp.maximum(jnp.dot(z, wa_ref[...]) + ba_ref[...], 0.0)
    out_ref[...] = jnp.maximum(
        jnp.dot(t, wb_ref[...]) + bb_ref[...], 0.0)


def _node_mlp(h, p0, p1, Wa, ba, Wb, bb, din):
    BN = 2000
    nblk = N // BN
    dh = din // 2
    full = lambda r, c: pl.BlockSpec((r, c), lambda i: (0, 0))
    return pl.pallas_call(
        _node_mlp_kernel,
        grid=(nblk,),
        in_specs=[
            pl.BlockSpec((BN, din), lambda i: (i, 0)),
            pl.BlockSpec((BN, dh), lambda i: (i, 0)),
            pl.BlockSpec((BN, dh), lambda i: (i, 0)),
            full(din, H), full(1, H), full(H, H), full(1, H),
        ],
        out_specs=pl.BlockSpec((BN, H), lambda i: (i, 0)),
        out_shape=jax.ShapeDtypeStruct((N, H), jnp.float32),
    )(h, p0, p1, Wa, ba.reshape(1, H), Wb, bb.reshape(1, H))


def _node_mlp2_kernel(h_ref, p0_ref, p1_ref, wa_ref, ba_ref, wb_ref, bb_ref,
                      we1x_ref, out_ref):
    z = h_ref[...] + jnp.concatenate([p0_ref[...], p1_ref[...]], axis=1)
    t = jnp.maximum(jnp.dot(z, wa_ref[...]) + ba_ref[...], 0.0)
    x2 = jnp.maximum(jnp.dot(t, wb_ref[...]) + bb_ref[...], 0.0)
    out_ref[...] = jnp.dot(x2, we1x_ref[...])


def _node_mlp2(h, p0, p1, Wa, ba, Wb, bb, We1x):
    BN = 2000
    nblk = N // BN
    full = lambda r, c: pl.BlockSpec((r, c), lambda i: (0, 0))
    return pl.pallas_call(
        _node_mlp2_kernel,
        grid=(nblk,),
        in_specs=[
            pl.BlockSpec((BN, H), lambda i: (i, 0)),
            pl.BlockSpec((BN, H // 2), lambda i: (i, 0)),
            pl.BlockSpec((BN, H // 2), lambda i: (i, 0)),
            full(H, H), full(1, H), full(H, H), full(1, H), full(H, H),
        ],
        out_specs=pl.BlockSpec((BN, H), lambda i: (i, 0)),
        out_shape=jax.ShapeDtypeStruct((N, H), jnp.float32),
    )(h, p0, p1, Wa, ba.reshape(1, H), Wb, bb.reshape(1, H), We1x)


def _edge_out_kernel(gg_ref, ea_ref, we1e_ref, be1_ref, we2_ref, be2_ref,
                     out_ref):
    g = jnp.dot(ea_ref[...], we1e_ref[...]) + be1_ref[...]
    r = jnp.maximum(gg_ref[...] + g, 0.0)
    out_ref[...] = jnp.dot(r, we2_ref[...]) + be2_ref[0, 0]


def _edge_out(G, edge_attr, We1e, be1, We2, be2):
    BE = 8000
    full = lambda r, c: pl.BlockSpec((r, c), lambda i: (0, 0))
    out = pl.pallas_call(
        _edge_out_kernel,
        grid=(E // BE,),
        in_specs=[
            pl.BlockSpec((BE, H), lambda i: (i, 0)),
            pl.BlockSpec((BE, DE), lambda i: (i, 0)),
            full(DE, H), full(1, H), full(H, 1), full(1, 1),
        ],
        out_specs=pl.BlockSpec((BE, 1), lambda i: (i, 0)),
        out_shape=jax.ShapeDtypeStruct((E, 1), jnp.float32),
    )(G, edge_attr, We1e, be1.reshape(1, H), We2, be2.reshape(1, 1))
    return out.reshape(-1)


def _sc_layer(hsplit, esplit, src, dst, dhalf, chunk):
    nch = EPT // chunk
    assert nch % 2 == 0
    mesh = plsc.VectorSubcoreMesh(core_axis_name="c", subcore_axis_name="s")

    @functools.partial(
        pl.kernel,
        out_type=jax.ShapeDtypeStruct((2 * NP, dhalf), jnp.float32),
        mesh=mesh,
        compiler_params=pltpu.CompilerParams(use_tc_tiling_on_sc=False),
        scratch_types=[
            pltpu.VMEM((EPT,), jnp.int32),
            pltpu.VMEM((2, chunk), jnp.int32),
            pltpu.VMEM((2, chunk, dhalf), jnp.float32),
            pltpu.VMEM((2, chunk, dhalf), jnp.float32),
            pltpu.VMEM_SHARED((NP, dhalf), jnp.float32),
            pltpu.SemaphoreType.DMA,
            pltpu.SemaphoreType.DMA,
            pltpu.SemaphoreType.DMA,
            pltpu.SemaphoreType.DMA,
        ],
    )
    def k(h_hbm, e_hbm, src_hbm, dst_hbm, out_hbm, sidx_all, didx2, xs2, es2,
          acc, dsem0, dsem1, ssem0, ssem1):
        cid = lax.axis_index("c")
        sid = lax.axis_index("s")
        dsems = (dsem0, dsem1)
        ssems = (ssem0, ssem1)
        tbase = sid * EPT

        pltpu.sync_copy(src_hbm.at[pl.ds(tbase, EPT)], sidx_all)
        off = cid * N

        def adj(i, _):
            sl = pl.ds(i * 16, 16)
            sidx_all[sl] = sidx_all[sl] + off
            return 0
        lax.fori_loop(0, EPT // 16, adj, 0)

        def zrow(i, _):
            for j in range(dhalf // 16):
                xs2[0, i, pl.ds(j * 16, 16)] = jnp.zeros((16,), jnp.float32)
            return 0
        lax.fori_loop(0, chunk, zrow, 0)
        zbase = sid * ROWS_PER_TILE
        done = 0
        while done < ROWS_PER_TILE:
            step = min(chunk, ROWS_PER_TILE - done)
            pltpu.sync_copy(xs2.at[0, pl.ds(0, step)],
                            acc.at[pl.ds(zbase + done, step)])
            done += step
        plsc.subcore_barrier()

        def gather_desc(kk, b, sem):
            return pltpu.make_async_copy(
                h_hbm.at[sidx_all.at[pl.ds(kk * chunk, chunk)]],
                xs2.at[b], sem)

        def erow_desc(kk, b, sem):
            return pltpu.make_async_copy(
                e_hbm.at[pl.ds(cid * E + tbase + kk * chunk, chunk)],
                es2.at[b], sem)

        def didx_desc(kk, b, sem):
            return pltpu.make_async_copy(
                dst_hbm.at[pl.ds(tbase + kk * chunk, chunk)],
                didx2.at[b], sem)

        def issue_data(kk, b):
            gather_desc(kk, b, dsems[b]).start()
            erow_desc(kk, b, dsems[b]).start()
            didx_desc(kk, b, dsems[b]).start()

        def wait_data(kk, b):
            gather_desc(kk, b, dsems[b]).wait()
            erow_desc(kk, b, dsems[b]).wait()
            didx_desc(kk, b, dsems[b]).wait()

        def scatter_desc(b):
            return pltpu.make_async_copy(
                xs2.at[b], acc.at[didx2.at[b]], ssems[b])

        def step(kk, b, o):
            @pl.when(kk >= 1)
            def _():
                scatter_desc(o).wait()

            @pl.when(kk + 1 < nch)
            def _():
                issue_data(kk + 1, o)
            wait_data(kk, b)

            def crow(i, _):
                for r in range(4):
                    for j in range(dhalf // 16):
                        sl = pl.ds(j * 16, 16)
                        xs2[b, 4 * i + r, sl] = jnp.maximum(
                            xs2[b, 4 * i + r, sl] + es2[b, 4 * i + r, sl],
                            0.0)
                return 0
            lax.fori_loop(0, chunk // 4, crow, 0)
            scatter_desc(b).start(add=True)

        issue_data(0, 0)

        def pair(p, _):
            step(2 * p, 0, 1)
            step(2 * p + 1, 1, 0)
            return 0
        lax.fori_loop(0, nch // 2, pair, 0)
        scatter_desc((nch - 1) % 2).wait()
        plsc.subcore_barrier()

        pltpu.sync_copy(
            acc.at[pl.ds(sid * ROWS_PER_TILE, ROWS_PER_TILE)],
            out_hbm.at[pl.ds(cid * NP + sid * ROWS_PER_TILE, ROWS_PER_TILE)])

    return k(hsplit, esplit, src, dst)


def _sc_gather(y, src, chunk):
    npt = E // (NUM_CORES * NUM_SUBCORES)
    nch = npt // chunk
    assert nch % 2 == 0
    mesh = plsc.VectorSubcoreMesh(core_axis_name="c", subcore_axis_name="s")

    @functools.partial(
        pl.kernel,
        out_type=jax.ShapeDtypeStruct((E, H), jnp.float32),
        mesh=mesh,
        compiler_params=pltpu.CompilerParams(use_tc_tiling_on_sc=False),
        scratch_types=[
            pltpu.VMEM((npt,), jnp.int32),
            pltpu.VMEM((2, chunk, H), jnp.float32),
            pltpu.SemaphoreType.DMA,
            pltpu.SemaphoreType.DMA,
            pltpu.SemaphoreType.DMA,
            pltpu.SemaphoreType.DMA,
        ],
    )
    def k(y_hbm, src_hbm, out_hbm, sidx_all, rows2, gsem0, gsem1, wsem0,
          wsem1):
        cid = lax.axis_index("c")
        sid = lax.axis_index("s")
        wid = cid * NUM_SUBCORES + sid
        tbase = wid * npt
        gsems = (gsem0, gsem1)
        wsems = (wsem0, wsem1)

        pltpu.sync_copy(src_hbm.at[pl.ds(tbase, npt)], sidx_all)

        def gather_desc(kk, b):
            return pltpu.make_async_copy(
                y_hbm.at[sidx_all.at[pl.ds(kk * chunk, chunk)]],
                rows2.at[b], gsems[b])

        def write_desc(kk, b):
            return pltpu.make_async_copy(
                rows2.at[b], out_hbm.at[pl.ds(tbase + kk * chunk, chunk)],
                wsems[b])

        def step(kk, b, o):
            @pl.when(kk + 1 < nch)
            def _():
                @pl.when(kk >= 1)
                def _():
                    write_desc(kk - 1, o).wait()
                gather_desc(kk + 1, o).start()
            gather_desc(kk, b).wait()
            write_desc(kk, b).start()

        gather_desc(0, 0).start()

        def pair(p, _):
            step(2 * p, 0, 1)
            step(2 * p + 1, 1, 0)
            return 0
        lax.fori_loop(0, nch // 2, pair, 0)
        write_desc(nch - 2, (nch - 2) % 2).wait()
        write_desc(nch - 1, (nch - 1) % 2).wait()

    return k(y, src)


def _split_cols(a, dhalf):
    return jnp.concatenate([a[:, :dhalf], a[:, dhalf:]], axis=0)


def kernel(x, edge_index, edge_attr, Wl1, bl1, W1a, b1a, W1b, b1b,
           Wl2, bl2, W2a, b2a, W2b, b2b, We1, be1, We2, be2):
    src = edge_index[0]
    dst = edge_index[1]
    We1x = We1[:H]
    We1e = We1[H:]

    e1s = _edge_lin(edge_attr, Wl1, bl1, D)
    e2s = _edge_lin(edge_attr, Wl2, bl2, H)

    p1 = _sc_layer(_split_cols(x, D // 2), e1s, src, dst, D // 2, 200)
    x1 = _node_mlp(x, p1[:N], p1[NP:NP + N], W1a, b1a, W1b, b1b, D)

    p2 = _sc_layer(_split_cols(x1, H // 2), e2s, src, dst, H // 2, 200)
    y2 = _node_mlp2(x1, p2[:N], p2[NP:NP + N], W2a, b2a, W2b, b2b, We1x)

    G = _sc_gather(y2, src, 200)
    return _edge_out(G, edge_attr, We1e, be1, We2, be2)


# speedup vs baseline: 1.1051x; 1.1051x over previous
import functools

import jax
import jax.numpy as jnp
from jax import lax
from jax.experimental import pallas as pl
from jax.experimental.pallas import tpu as pltpu
from jax.experimental.pallas import tpu_sc as plsc

N = 10000
E = 320000
D = 128
DE = 16
H = 64

NUM_CORES = 2
NUM_SUBCORES = 16
EPT = E // NUM_SUBCORES
NP = 10240
ROWS_PER_TILE = NP // NUM_SUBCORES

_HI = lax.Precision.HIGHEST


def _edge_lin_kernel(ea_ref, wl1_ref, bl1_ref, wl2_ref, bl2_ref,
                     e1_ref, e2_ref):
    ea = ea_ref[...]
    e1_ref[...] = jnp.dot(ea, wl1_ref[0]) + bl1_ref[0]
    e2_ref[...] = jnp.dot(ea, wl2_ref[0]) + bl2_ref[0]


def _edge_lin2(edge_attr, Wl1, bl1, Wl2, bl2):
    BE = 8000
    nblk = E // BE
    dh1, dh2 = D // 2, H // 2
    wl1s = Wl1.reshape(DE, 2, dh1).transpose(1, 0, 2)
    bl1s = bl1.reshape(2, 1, dh1)
    wl2s = Wl2.reshape(DE, 2, dh2).transpose(1, 0, 2)
    bl2s = bl2.reshape(2, 1, dh2)
    return pl.pallas_call(
        _edge_lin_kernel,
        grid=(2, nblk),
        in_specs=[
            pl.BlockSpec((BE, DE), lambda j, i: (i, 0)),
            pl.BlockSpec((1, DE, dh1), lambda j, i: (j, 0, 0)),
            pl.BlockSpec((1, 1, dh1), lambda j, i: (j, 0, 0)),
            pl.BlockSpec((1, DE, dh2), lambda j, i: (j, 0, 0)),
            pl.BlockSpec((1, 1, dh2), lambda j, i: (j, 0, 0)),
        ],
        out_specs=[
            pl.BlockSpec((BE, dh1), lambda j, i: (j * nblk + i, 0)),
            pl.BlockSpec((BE, dh2), lambda j, i: (j * nblk + i, 0)),
        ],
        out_shape=[
            jax.ShapeDtypeStruct((2 * E, dh1), jnp.float32),
            jax.ShapeDtypeStruct((2 * E, dh2), jnp.float32),
        ],
    )(edge_attr, wl1s, bl1s, wl2s, bl2s)


def _node_mlp_kernel(h_ref, p0_ref, p1_ref, wa_ref, ba_ref, wb_ref, bb_ref,
                     out_ref):
    z = h_ref[...] + jnp.concatenate([p0_ref[...], p1_ref[...]], axis=1)
    t = jnp.maximum(jnp.dot(z, wa_ref[...]) + ba_ref[...], 0.0)
    out_ref[...] = jnp.maximum(
        jnp.dot(t, wb_ref[...]) + bb_ref[...], 0.0)


def _node_mlp(h, p0, p1, Wa, ba, Wb, bb, din):
    BN = 2000
    nblk = N // BN
    dh = din // 2
    full = lambda r, c: pl.BlockSpec((r, c), lambda i: (0, 0))
    return pl.pallas_call(
        _node_mlp_kernel,
        grid=(nblk,),
        in_specs=[
            pl.BlockSpec((BN, din), lambda i: (i, 0)),
            pl.BlockSpec((BN, dh), lambda i: (i, 0)),
            pl.BlockSpec((BN, dh), lambda i: (i, 0)),
            full(din, H), full(1, H), full(H, H), full(1, H),
        ],
        out_specs=pl.BlockSpec((BN, H), lambda i: (i, 0)),
        out_shape=jax.ShapeDtypeStruct((N, H), jnp.float32),
    )(h, p0, p1, Wa, ba.reshape(1, H), Wb, bb.reshape(1, H))


def _node_mlp2_kernel(h_ref, p0_ref, p1_ref, wa_ref, ba_ref, wb_ref, bb_ref,
                      we1x_ref, out_ref):
    z = h_ref[...] + jnp.concatenate([p0_ref[...], p1_ref[...]], axis=1)
    t = jnp.maximum(jnp.dot(z, wa_ref[...]) + ba_ref[...], 0.0)
    x2 = jnp.maximum(jnp.dot(t, wb_ref[...]) + bb_ref[...], 0.0)
    out_ref[...] = jnp.dot(x2, we1x_ref[...])


def _node_mlp2(h, p0, p1, Wa, ba, Wb, bb, We1x):
    BN = 2000
    nblk = N // BN
    full = lambda r, c: pl.BlockSpec((r, c), lambda i: (0, 0))
    return pl.pallas_call(
        _node_mlp2_kernel,
        grid=(nblk,),
        in_specs=[
            pl.BlockSpec((BN, H), lambda i: (i, 0)),
            pl.BlockSpec((BN, H // 2), lambda i: (i, 0)),
            pl.BlockSpec((BN, H // 2), lambda i: (i, 0)),
            full(H, H), full(1, H), full(H, H), full(1, H), full(H, H),
        ],
        out_specs=pl.BlockSpec((BN, H), lambda i: (i, 0)),
        out_shape=jax.ShapeDtypeStruct((N, H), jnp.float32),
    )(h, p0, p1, Wa, ba.reshape(1, H), Wb, bb.reshape(1, H), We1x)


def _edge_out_kernel(gg_ref, ea_ref, we1e_ref, be1_ref, we2_ref, be2_ref,
                     out_ref):
    g = jnp.dot(ea_ref[...], we1e_ref[...]) + be1_ref[...]
    r = jnp.maximum(gg_ref[...] + g, 0.0)
    out_ref[...] = jnp.dot(r, we2_ref[...]) + be2_ref[0, 0]


def _edge_out(G, edge_attr, We1e, be1, We2, be2):
    BE = 8000
    full = lambda r, c: pl.BlockSpec((r, c), lambda i: (0, 0))
    out = pl.pallas_call(
        _edge_out_kernel,
        grid=(E // BE,),
        in_specs=[
            pl.BlockSpec((BE, H), lambda i: (i, 0)),
            pl.BlockSpec((BE, DE), lambda i: (i, 0)),
            full(DE, H), full(1, H), full(H, 1), full(1, 1),
        ],
        out_specs=pl.BlockSpec((BE, 1), lambda i: (i, 0)),
        out_shape=jax.ShapeDtypeStruct((E, 1), jnp.float32),
    )(G, edge_attr, We1e, be1.reshape(1, H), We2, be2.reshape(1, 1))
    return out.reshape(-1)


def _sc_layer(hsplit, esplit, src, dst, dhalf, chunk):
    nch = EPT // chunk
    assert nch % 2 == 0
    mesh = plsc.VectorSubcoreMesh(core_axis_name="c", subcore_axis_name="s")

    @functools.partial(
        pl.kernel,
        out_type=jax.ShapeDtypeStruct((2 * NP, dhalf), jnp.float32),
        mesh=mesh,
        compiler_params=pltpu.CompilerParams(use_tc_tiling_on_sc=False),
        scratch_types=[
            pltpu.VMEM((EPT,), jnp.int32),
            pltpu.VMEM((2, chunk), jnp.int32),
            pltpu.VMEM((2, chunk, dhalf), jnp.float32),
            pltpu.VMEM((2, chunk, dhalf), jnp.float32),
            pltpu.VMEM_SHARED((NP, dhalf), jnp.float32),
            pltpu.SemaphoreType.DMA,
            pltpu.SemaphoreType.DMA,
            pltpu.SemaphoreType.DMA,
            pltpu.SemaphoreType.DMA,
        ],
    )
    def k(h_hbm, e_hbm, src_hbm, dst_hbm, out_hbm, sidx_all, didx2, xs2, es2,
          acc, dsem0, dsem1, ssem0, ssem1):
        cid = lax.axis_index("c")
        sid = lax.axis_index("s")
        dsems = (dsem0, dsem1)
        ssems = (ssem0, ssem1)
        tbase = sid * EPT

        pltpu.sync_copy(src_hbm.at[pl.ds(tbase, EPT)], sidx_all)
        off = cid * N

        def adj(i, _):
            sl = pl.ds(i * 16, 16)
            sidx_all[sl] = sidx_all[sl] + off
            return 0
        lax.fori_loop(0, EPT // 16, adj, 0)

        def zrow(i, _):
            for j in range(dhalf // 16):
                xs2[0, i, pl.ds(j * 16, 16)] = jnp.zeros((16,), jnp.float32)
            return 0
        lax.fori_loop(0, chunk, zrow, 0)
        zbase = sid * ROWS_PER_TILE
        done = 0
        while done < ROWS_PER_TILE:
            step = min(chunk, ROWS_PER_TILE - done)
            pltpu.sync_copy(xs2.at[0, pl.ds(0, step)],
                            acc.at[pl.ds(zbase + done, step)])
            done += step
        plsc.subcore_barrier()

        def gather_desc(kk, b, sem):
            return pltpu.make_async_copy(
                h_hbm.at[sidx_all.at[pl.ds(kk * chunk, chunk)]],
                xs2.at[b], sem)

        def erow_desc(kk, b, sem):
            return pltpu.make_async_copy(
                e_hbm.at[pl.ds(cid * E + tbase + kk * chunk, chunk)],
                es2.at[b], sem)

        def didx_desc(kk, b, sem):
            return pltpu.make_async_copy(
                dst_hbm.at[pl.ds(tbase + kk * chunk, chunk)],
                didx2.at[b], sem)

        def issue_data(kk, b):
            gather_desc(kk, b, dsems[b]).start()
            erow_desc(kk, b, dsems[b]).start()
            didx_desc(kk, b, dsems[b]).start()

        def wait_data(kk, b):
            gather_desc(kk, b, dsems[b]).wait()
            erow_desc(kk, b, dsems[b]).wait()
            didx_desc(kk, b, dsems[b]).wait()

        def scatter_desc(b):
            return pltpu.make_async_copy(
                xs2.at[b], acc.at[didx2.at[b]], ssems[b])

        def step(kk, b, o):
            @pl.when(kk >= 1)
            def _():
                scatter_desc(o).wait()

            @pl.when(kk + 1 < nch)
            def _():
                issue_data(kk + 1, o)
            wait_data(kk, b)

            def crow(i, _):
                for r in range(4):
                    for j in range(dhalf // 16):
                        sl = pl.ds(j * 16, 16)
                        xs2[b, 4 * i + r, sl] = jnp.maximum(
                            xs2[b, 4 * i + r, sl] + es2[b, 4 * i + r, sl],
                            0.0)
                return 0
            lax.fori_loop(0, chunk // 4, crow, 0)
            scatter_desc(b).start(add=True)

        issue_data(0, 0)

        def pair(p, _):
            step(2 * p, 0, 1)
            step(2 * p + 1, 1, 0)
            return 0
        lax.fori_loop(0, nch // 2, pair, 0)
        scatter_desc((nch - 1) % 2).wait()
        plsc.subcore_barrier()

        pltpu.sync_copy(
            acc.at[pl.ds(sid * ROWS_PER_TILE, ROWS_PER_TILE)],
            out_hbm.at[pl.ds(cid * NP + sid * ROWS_PER_TILE, ROWS_PER_TILE)])

    return k(hsplit, esplit, src, dst)


def _sc_gather(y, src, chunk):
    npt = E // (NUM_CORES * NUM_SUBCORES)
    nch = npt // chunk
    assert nch % 2 == 0
    mesh = plsc.VectorSubcoreMesh(core_axis_name="c", subcore_axis_name="s")

    @functools.partial(
        pl.kernel,
        out_type=jax.ShapeDtypeStruct((E, H), jnp.float32),
        mesh=mesh,
        compiler_params=pltpu.CompilerParams(use_tc_tiling_on_sc=False),
        scratch_types=[
            pltpu.VMEM((npt,), jnp.int32),
            pltpu.VMEM((2, chunk, H), jnp.float32),
            pltpu.SemaphoreType.DMA,
            pltpu.SemaphoreType.DMA,
            pltpu.SemaphoreType.DMA,
            pltpu.SemaphoreType.DMA,
        ],
    )
    def k(y_hbm, src_hbm, out_hbm, sidx_all, rows2, gsem0, gsem1, wsem0,
          wsem1):
        cid = lax.axis_index("c")
        sid = lax.axis_index("s")
        wid = cid * NUM_SUBCORES + sid
        tbase = wid * npt
        gsems = (gsem0, gsem1)
        wsems = (wsem0, wsem1)

        pltpu.sync_copy(src_hbm.at[pl.ds(tbase, npt)], sidx_all)

        def gather_desc(kk, b):
            return pltpu.make_async_copy(
                y_hbm.at[sidx_all.at[pl.ds(kk * chunk, chunk)]],
                rows2.at[b], gsems[b])

        def write_desc(kk, b):
            return pltpu.make_async_copy(
                rows2.at[b], out_hbm.at[pl.ds(tbase + kk * chunk, chunk)],
                wsems[b])

        def step(kk, b, o):
            @pl.when(kk + 1 < nch)
            def _():
                @pl.when(kk >= 1)
                def _():
                    write_desc(kk - 1, o).wait()
                gather_desc(kk + 1, o).start()
            gather_desc(kk, b).wait()
            write_desc(kk, b).start()

        gather_desc(0, 0).start()

        def pair(p, _):
            step(2 * p, 0, 1)
            step(2 * p + 1, 1, 0)
            return 0
        lax.fori_loop(0, nch // 2, pair, 0)
        write_desc(nch - 2, (nch - 2) % 2).wait()
        write_desc(nch - 1, (nch - 1) % 2).wait()

    return k(y, src)


def _split_cols(a, dhalf):
    return jnp.concatenate([a[:, :dhalf], a[:, dhalf:]], axis=0)


def kernel(x, edge_index, edge_attr, Wl1, bl1, W1a, b1a, W1b, b1b,
           Wl2, bl2, W2a, b2a, W2b, b2b, We1, be1, We2, be2):
    src = edge_index[0]
    dst = edge_index[1]
    We1x = We1[:H]
    We1e = We1[H:]

    e1s, e2s = _edge_lin2(edge_attr, Wl1, bl1, Wl2, bl2)

    p1 = _sc_layer(_split_cols(x, D // 2), e1s, src, dst, D // 2, 200)
    x1 = _node_mlp(x, p1[:N], p1[NP:NP + N], W1a, b1a, W1b, b1b, D)

    p2 = _sc_layer(_split_cols(x1, H // 2), e2s, src, dst, H // 2, 200)
    y2 = _node_mlp2(x1, p2[:N], p2[NP:NP + N], W2a, b2a, W2b, b2b, We1x)

    G = _sc_gather(y2, src, 200)
    return _edge_out(G, edge_attr, We1e, be1, We2, be2)


# speedup vs baseline: 1.6670x; 1.5085x over previous
import functools

import jax
import jax.numpy as jnp
from jax import lax
from jax.experimental import pallas as pl
from jax.experimental.pallas import tpu as pltpu
from jax.experimental.pallas import tpu_sc as plsc

N = 10000
E = 320000
D = 128
DE = 16
H = 64

NUM_CORES = 2
NUM_SUBCORES = 16
EPT = E // NUM_SUBCORES
NP = 10240
ROWS_PER_TILE = NP // NUM_SUBCORES

_HI = lax.Precision.HIGHEST


def _edge_lin_kernel(ea_ref, wl1_ref, bl1_ref, wl2_ref, bl2_ref,
                     e1_ref, e2_ref):
    ea = ea_ref[...]
    e1_ref[...] = jnp.dot(ea, wl1_ref[0]) + bl1_ref[0]
    e2_ref[...] = jnp.dot(ea, wl2_ref[0]) + bl2_ref[0]


def _blockdiag(W, dh):
    z = jnp.zeros((8, DE, 8, dh), jnp.float32)
    idx = jnp.arange(8)
    z = z.at[idx, :, idx, :].set(W)
    return z.reshape(8 * DE, 8 * dh)


def _edge_lin2(eac, Wl1, bl1, Wl2, bl2):
    BE = 8000
    nblk = E // BE
    dh1, dh2 = D // 2, H // 2
    wl1s = jnp.stack([_blockdiag(Wl1[:, :dh1], dh1),
                      _blockdiag(Wl1[:, dh1:], dh1)])
    bl1s = jnp.stack([jnp.tile(bl1[:dh1], 8), jnp.tile(bl1[dh1:], 8)])
    bl1s = bl1s.reshape(2, 1, 8 * dh1)
    wl2s = jnp.stack([_blockdiag(Wl2[:, :dh2], dh2),
                      _blockdiag(Wl2[:, dh2:], dh2)])
    bl2s = jnp.stack([jnp.tile(bl2[:dh2], 8), jnp.tile(bl2[dh2:], 8)])
    bl2s = bl2s.reshape(2, 1, 8 * dh2)
    return pl.pallas_call(
        _edge_lin_kernel,
        grid=(2, nblk),
        in_specs=[
            pl.BlockSpec((BE // 8, 8 * DE), lambda j, i: (i, 0)),
            pl.BlockSpec((1, 8 * DE, 8 * dh1), lambda j, i: (j, 0, 0)),
            pl.BlockSpec((1, 1, 8 * dh1), lambda j, i: (j, 0, 0)),
            pl.BlockSpec((1, 8 * DE, 8 * dh2), lambda j, i: (j, 0, 0)),
            pl.BlockSpec((1, 1, 8 * dh2), lambda j, i: (j, 0, 0)),
        ],
        out_specs=[
            pl.BlockSpec((BE // 8, 8 * dh1), lambda j, i: (j * nblk + i, 0)),
            pl.BlockSpec((BE // 8, 8 * dh2), lambda j, i: (j * nblk + i, 0)),
        ],
        out_shape=[
            jax.ShapeDtypeStruct((2 * E // 8, 8 * dh1), jnp.float32),
            jax.ShapeDtypeStruct((2 * E // 8, 8 * dh2), jnp.float32),
        ],
    )(eac, wl1s, bl1s, wl2s, bl2s)


def _node_mlp_kernel(h_ref, p0_ref, p1_ref, wa_ref, ba_ref, wb_ref, bb_ref,
                     out_ref):
    z = h_ref[...] + jnp.concatenate([p0_ref[...], p1_ref[...]], axis=1)
    t = jnp.maximum(jnp.dot(z, wa_ref[...]) + ba_ref[...], 0.0)
    out_ref[...] = jnp.maximum(
        jnp.dot(t, wb_ref[...]) + bb_ref[...], 0.0)


def _node_mlp(h, p0, p1, Wa, ba, Wb, bb, din):
    BN = 2000
    nblk = N // BN
    dh = din // 2
    full = lambda r, c: pl.BlockSpec((r, c), lambda i: (0, 0))
    return pl.pallas_call(
        _node_mlp_kernel,
        grid=(nblk,),
        in_specs=[
            pl.BlockSpec((BN, din), lambda i: (i, 0)),
            pl.BlockSpec((BN, dh), lambda i: (i, 0)),
            pl.BlockSpec((BN, dh), lambda i: (i, 0)),
            full(din, H), full(1, H), full(H, H), full(1, H),
        ],
        out_specs=pl.BlockSpec((BN, H), lambda i: (i, 0)),
        out_shape=jax.ShapeDtypeStruct((N, H), jnp.float32),
    )(h, p0, p1, Wa, ba.reshape(1, H), Wb, bb.reshape(1, H))


def _node_mlp2_kernel(h_ref, p0_ref, p1_ref, wa_ref, ba_ref, wb_ref, bb_ref,
                      we1x_ref, out_ref):
    z = h_ref[...] + jnp.concatenate([p0_ref[...], p1_ref[...]], axis=1)
    t = jnp.maximum(jnp.dot(z, wa_ref[...]) + ba_ref[...], 0.0)
    x2 = jnp.maximum(jnp.dot(t, wb_ref[...]) + bb_ref[...], 0.0)
    out_ref[...] = jnp.dot(x2, we1x_ref[...])


def _node_mlp2(h, p0, p1, Wa, ba, Wb, bb, We1x):
    BN = 2000
    nblk = N // BN
    full = lambda r, c: pl.BlockSpec((r, c), lambda i: (0, 0))
    return pl.pallas_call(
        _node_mlp2_kernel,
        grid=(nblk,),
        in_specs=[
            pl.BlockSpec((BN, H), lambda i: (i, 0)),
            pl.BlockSpec((BN, H // 2), lambda i: (i, 0)),
            pl.BlockSpec((BN, H // 2), lambda i: (i, 0)),
            full(H, H), full(1, H), full(H, H), full(1, H), full(H, H),
        ],
        out_specs=pl.BlockSpec((BN, H), lambda i: (i, 0)),
        out_shape=jax.ShapeDtypeStruct((N, H), jnp.float32),
    )(h, p0, p1, Wa, ba.reshape(1, H), Wb, bb.reshape(1, H), We1x)


def _edge_out_kernel(gg_ref, ea_ref, we1e_ref, be1_ref, we2_ref, be2_ref,
                     out_ref):
    g = jnp.dot(ea_ref[...], we1e_ref[...]) + be1_ref[...]
    r = jnp.maximum(gg_ref[...] + g, 0.0)
    out_ref[...] = jnp.dot(r, we2_ref[...]) + be2_ref[0, 0]


def _edge_out(Gp, eac, We1e, be1, We2, be2):
    BE = 8000
    full = lambda r, c: pl.BlockSpec((r, c), lambda i: (0, 0))
    we1ebd = _blockdiag(We1e, H)
    be1t = jnp.tile(be1, 8).reshape(1, 8 * H)
    we2bd = jnp.zeros((8, H, 8), jnp.float32)
    idx = jnp.arange(8)
    we2bd = we2bd.at[idx, :, idx].set(We2[:, 0]).reshape(8 * H, 8)
    out = pl.pallas_call(
        _edge_out_kernel,
        grid=(E // BE,),
        in_specs=[
            pl.BlockSpec((BE // 8, 8 * H), lambda i: (i, 0)),
            pl.BlockSpec((BE // 8, 8 * DE), lambda i: (i, 0)),
            full(8 * DE, 8 * H), full(1, 8 * H), full(8 * H, 8), full(1, 1),
        ],
        out_specs=pl.BlockSpec((BE // 8, 8), lambda i: (i, 0)),
        out_shape=jax.ShapeDtypeStruct((E // 8, 8), jnp.float32),
    )(Gp, eac, we1ebd, be1t, we2bd, be2.reshape(1, 1))
    return out.reshape(-1)


def _sc_layer(hsplit, esplit, src, dst, dhalf, chunk):
    nch = EPT // chunk
    assert nch % 2 == 0
    mesh = plsc.VectorSubcoreMesh(core_axis_name="c", subcore_axis_name="s")

    @functools.partial(
        pl.kernel,
        out_type=jax.ShapeDtypeStruct((2 * NP, dhalf), jnp.float32),
        mesh=mesh,
        compiler_params=pltpu.CompilerParams(use_tc_tiling_on_sc=False),
        scratch_types=[
            pltpu.VMEM((EPT,), jnp.int32),
            pltpu.VMEM((2, chunk), jnp.int32),
            pltpu.VMEM((2, chunk, dhalf), jnp.float32),
            pltpu.VMEM((2, chunk // 8, 8 * dhalf), jnp.float32),
            pltpu.VMEM_SHARED((NP, dhalf), jnp.float32),
            pltpu.SemaphoreType.DMA,
            pltpu.SemaphoreType.DMA,
            pltpu.SemaphoreType.DMA,
            pltpu.SemaphoreType.DMA,
        ],
    )
    def k(h_hbm, e_hbm, src_hbm, dst_hbm, out_hbm, sidx_all, didx2, xs2, es2,
          acc, dsem0, dsem1, ssem0, ssem1):
        cid = lax.axis_index("c")
        sid = lax.axis_index("s")
        dsems = (dsem0, dsem1)
        ssems = (ssem0, ssem1)
        tbase = sid * EPT

        pltpu.sync_copy(src_hbm.at[pl.ds(tbase, EPT)], sidx_all)
        off = cid * N

        def adj(i, _):
            sl = pl.ds(i * 16, 16)
            sidx_all[sl] = sidx_all[sl] + off
            return 0
        lax.fori_loop(0, EPT // 16, adj, 0)

        def zrow(i, _):
            for j in range(dhalf // 16):
                xs2[0, i, pl.ds(j * 16, 16)] = jnp.zeros((16,), jnp.float32)
            return 0
        lax.fori_loop(0, chunk, zrow, 0)
        zbase = sid * ROWS_PER_TILE
        done = 0
        while done < ROWS_PER_TILE:
            step = min(chunk, ROWS_PER_TILE - done)
            pltpu.sync_copy(xs2.at[0, pl.ds(0, step)],
                            acc.at[pl.ds(zbase + done, step)])
            done += step
        plsc.subcore_barrier()

        def gather_desc(kk, b, sem):
            return pltpu.make_async_copy(
                h_hbm.at[sidx_all.at[pl.ds(kk * chunk, chunk)]],
                xs2.at[b], sem)

        def erow_desc(kk, b, sem):
            return pltpu.make_async_copy(
                e_hbm.at[pl.ds((cid * E + tbase + kk * chunk) // 8,
                               chunk // 8)],
                es2.at[b], sem)

        def didx_desc(kk, b, sem):
            return pltpu.make_async_copy(
                dst_hbm.at[pl.ds(tbase + kk * chunk, chunk)],
                didx2.at[b], sem)

        def issue_data(kk, b):
            gather_desc(kk, b, dsems[b]).start()
            erow_desc(kk, b, dsems[b]).start()
            didx_desc(kk, b, dsems[b]).start()

        def wait_data(kk, b):
            gather_desc(kk, b, dsems[b]).wait()
            erow_desc(kk, b, dsems[b]).wait()
            didx_desc(kk, b, dsems[b]).wait()

        def scatter_desc(b):
            return pltpu.make_async_copy(
                xs2.at[b], acc.at[didx2.at[b]], ssems[b])

        def step(kk, b, o):
            @pl.when(kk >= 1)
            def _():
                scatter_desc(o).wait()

            @pl.when(kk + 1 < nch)
            def _():
                issue_data(kk + 1, o)
            wait_data(kk, b)

            def crow(i, _):
                for r in range(8):
                    for j in range(dhalf // 16):
                        sl = pl.ds(j * 16, 16)
                        el = pl.ds(r * dhalf + j * 16, 16)
                        xs2[b, 8 * i + r, sl] = jnp.maximum(
                            xs2[b, 8 * i + r, sl] + es2[b, i, el], 0.0)
                return 0
            lax.fori_loop(0, chunk // 8, crow, 0)
            scatter_desc(b).start(add=True)

        issue_data(0, 0)

        def pair(p, _):
            step(2 * p, 0, 1)
            step(2 * p + 1, 1, 0)
            return 0
        lax.fori_loop(0, nch // 2, pair, 0)
        scatter_desc((nch - 1) % 2).wait()
        plsc.subcore_barrier()

        pltpu.sync_copy(
            acc.at[pl.ds(sid * ROWS_PER_TILE, ROWS_PER_TILE)],
            out_hbm.at[pl.ds(cid * NP + sid * ROWS_PER_TILE, ROWS_PER_TILE)])

    return k(hsplit, esplit, src, dst)


def _sc_gather(y, src, chunk):
    npt = E // (NUM_CORES * NUM_SUBCORES)
    nch = npt // chunk
    assert nch % 2 == 0
    mesh = plsc.VectorSubcoreMesh(core_axis_name="c", subcore_axis_name="s")

    @functools.partial(
        pl.kernel,
        out_type=jax.ShapeDtypeStruct((E, H), jnp.float32),
        mesh=mesh,
        compiler_params=pltpu.CompilerParams(use_tc_tiling_on_sc=False),
        scratch_types=[
            pltpu.VMEM((npt,), jnp.int32),
            pltpu.VMEM((2, chunk, H), jnp.float32),
            pltpu.SemaphoreType.DMA,
            pltpu.SemaphoreType.DMA,
            pltpu.SemaphoreType.DMA,
            pltpu.SemaphoreType.DMA,
        ],
    )
    def k(y_hbm, src_hbm, out_hbm, sidx_all, rows2, gsem0, gsem1, wsem0,
          wsem1):
        cid = lax.axis_index("c")
        sid = lax.axis_index("s")
        wid = cid * NUM_SUBCORES + sid
        tbase = wid * npt
        gsems = (gsem0, gsem1)
        wsems = (wsem0, wsem1)

        pltpu.sync_copy(src_hbm.at[pl.ds(tbase, npt)], sidx_all)

        def gather_desc(kk, b):
            return pltpu.make_async_copy(
                y_hbm.at[sidx_all.at[pl.ds(kk * chunk, chunk)]],
                rows2.at[b], gsems[b])

        def write_desc(kk, b):
            return pltpu.make_async_copy(
                rows2.at[b], out_hbm.at[pl.ds(tbase + kk * chunk, chunk)],
                wsems[b])

        def step(kk, b, o):
            @pl.when(kk + 1 < nch)
            def _():
                @pl.when(kk >= 1)
                def _():
                    write_desc(kk - 1, o).wait()
                gather_desc(kk + 1, o).start()
            gather_desc(kk, b).wait()
            write_desc(kk, b).start()

        gather_desc(0, 0).start()

        def pair(p, _):
            step(2 * p, 0, 1)
            step(2 * p + 1, 1, 0)
            return 0
        lax.fori_loop(0, nch // 2, pair, 0)
        write_desc(nch - 2, (nch - 2) % 2).wait()
        write_desc(nch - 1, (nch - 1) % 2).wait()

    return k(y, src)


def _split_cols(a, dhalf):
    return jnp.concatenate([a[:, :dhalf], a[:, dhalf:]], axis=0)


def kernel(x, edge_index, edge_attr, Wl1, bl1, W1a, b1a, W1b, b1b,
           Wl2, bl2, W2a, b2a, W2b, b2b, We1, be1, We2, be2):
    src = edge_index[0]
    dst = edge_index[1]
    We1x = We1[:H]
    We1e = We1[H:]

    eac = edge_attr.reshape(E // 8, 8 * DE)
    e1s, e2s = _edge_lin2(eac, Wl1, bl1, Wl2, bl2)

    p1 = _sc_layer(_split_cols(x, D // 2), e1s, src, dst, D // 2, 200)
    x1 = _node_mlp(x, p1[:N], p1[NP:NP + N], W1a, b1a, W1b, b1b, D)

    p2 = _sc_layer(_split_cols(x1, H // 2), e2s, src, dst, H // 2, 200)
    y2 = _node_mlp2(x1, p2[:N], p2[NP:NP + N], W2a, b2a, W2b, b2b, We1x)

    G = _sc_gather(y2, src, 200)
    return _edge_out(G.reshape(E // 8, 8 * H), eac, We1e, be1, We2, be2)
